# Initial kernel scaffold; baseline (speedup 1.0000x reference)
import jax
import jax.numpy as jnp
from jax.experimental import pallas as pl


def kernel(x, edge_index, edge_type, rel_emb, W_lin, W_attn):
    raise NotImplementedError("write your pallas kernel here")


# speedup vs baseline: 9.1734x; 9.1734x over previous
import jax
import jax.numpy as jnp
from jax import lax
from jax.experimental import pallas as pl
from jax.experimental.pallas import tpu as pltpu
from jax.experimental.pallas import tpu_sc as plsc

N_NODES = 10000
N_EDGES = 320000
D = 128
DH = D // 2
N_REL = 50

NC = 2
NS = 16
CHUNK = 128
CHUNKS_PER_W = 157
EPW = CHUNKS_PER_W * CHUNK
E_PAD = NS * EPW
N_PAD = 10016
ACC_ROWS = 10240
ROWS_PER_TILE = ACC_ROWS // NS


def _tc_prep(x_ref, wl_ref, wa_ref, rel_ref, xt_ref, s_ref, r_ref):
    xv = x_ref[...]
    xt = lax.dot_general(
        xv, wl_ref[...], (((1,), (1,)), ((), ())),
        preferred_element_type=jnp.float32)
    xt_ref[0:N_PAD, :] = xt[:, 0:DH]
    xt_ref[N_PAD:2 * N_PAD, :] = xt[:, DH:D]
    wa = wa_ref[...]
    s_ref[...] = lax.dot_general(
        xv, wa, (((1,), (1,)), ((), ())), preferred_element_type=jnp.float32)
    r_ref[...] = lax.dot_general(
        rel_ref[...], wa, (((1,), (1,)), ((), ())),
        preferred_element_type=jnp.float32)


def _tc_combine(p_ref, o_ref):
    o_ref[...] = jnp.maximum(
        jnp.concatenate([p_ref[0], p_ref[1]], axis=1), 0.0)


def _sc_edges(xt_hbm, s_hbm, r_hbm, src_hbm, tgt_hbm, typ_hbm, part_hbm,
              src_v, tgt_v, typ_v, s_v, r_v, rows_v, acc, sem):
    c = lax.axis_index("c")
    s = lax.axis_index("s")

    @pl.loop(0, CHUNK)
    def _zero_rows(i):
        zero16 = jnp.zeros((16,), jnp.float32)
        for h in range(DH // 16):
            rows_v[i, pl.ds(h * 16, 16)] = zero16

    for b in range(ROWS_PER_TILE // CHUNK):
        pltpu.sync_copy(rows_v, acc.at[pl.ds((s * 5 + b) * CHUNK, CHUNK)])
    plsc.subcore_barrier()

    pltpu.sync_copy(src_hbm.at[c, s], src_v)
    pltpu.sync_copy(tgt_hbm.at[s], tgt_v)
    pltpu.sync_copy(typ_hbm.at[s], typ_v)
    pltpu.sync_copy(s_hbm, s_v)
    pltpu.sync_copy(r_hbm, r_v)

    @pl.loop(0, CHUNKS_PER_W)
    def _chunk(j):
        pltpu.async_copy(xt_hbm.at[src_v.at[j]], rows_v, sem).wait()
        for k in range(CHUNK // 16):
            sl = pl.ds(k * 16, 16)
            idx16 = src_v[j, sl]
            typ16 = typ_v[j, sl]
            sv = plsc.load_gather(s_v, [idx16])
            rv = plsc.load_gather(r_v, [typ16])
            a16 = 1.0 / (1.0 + jnp.exp(-(sv + rv)))
            for l in range(16):
                e = k * 16 + l
                a = lax.broadcast_in_dim(a16[l], (16,), ())
                for h in range(DH // 16):
                    fsl = pl.ds(h * 16, 16)
                    rows_v[e, fsl] = rows_v[e, fsl] * a

        pltpu.sync_copy(rows_v, acc.at[tgt_v.at[j]], add=True)

    plsc.subcore_barrier()
    pltpu.sync_copy(acc.at[pl.ds(s * ROWS_PER_TILE, ROWS_PER_TILE)],
                    part_hbm.at[c, pl.ds(s * ROWS_PER_TILE, ROWS_PER_TILE)])


@jax.jit
def _run(x, edge_index, edge_type, rel_emb, W_lin, W_attn):
    src = edge_index[0].astype(jnp.int32)
    tgt = edge_index[1].astype(jnp.int32)
    typ = edge_type.astype(jnp.int32)

    pad = E_PAD - N_EDGES
    src = jnp.concatenate([src, jnp.full((pad,), N_NODES, jnp.int32)])
    tgt = jnp.concatenate([tgt, jnp.zeros((pad,), jnp.int32)])
    typ = jnp.concatenate([typ, jnp.zeros((pad,), jnp.int32)])
    src = src.reshape(NS, CHUNKS_PER_W, CHUNK)
    tgt = tgt.reshape(NS, CHUNKS_PER_W, CHUNK)
    typ = typ.reshape(NS, CHUNKS_PER_W, CHUNK)
    src_off = src[None] + (jnp.arange(NC, dtype=jnp.int32) * N_PAD)[
        :, None, None, None]

    x_pad = jnp.concatenate(
        [x, jnp.zeros((N_PAD - N_NODES, D), jnp.float32)], axis=0)
    rel_pad = jnp.concatenate(
        [rel_emb, jnp.zeros((64 - N_REL, D), jnp.float32)], axis=0)

    xt_split, s_pad, r_pad = pl.pallas_call(
        _tc_prep,
        out_shape=[
            jax.ShapeDtypeStruct((NC * N_PAD, DH), jnp.float32),
            jax.ShapeDtypeStruct((N_PAD, 1), jnp.float32),
            jax.ShapeDtypeStruct((64, 1), jnp.float32),
        ],
    )(x_pad, W_lin, W_attn, rel_pad)

    s2 = jnp.concatenate([s_pad.reshape(N_PAD), s_pad.reshape(N_PAD)])
    r1 = r_pad.reshape(64)

    mesh = plsc.VectorSubcoreMesh(
        core_axis_name="c", subcore_axis_name="s",
        num_cores=NC, num_subcores=NS)
    sc_call = pl.kernel(
        _sc_edges,
        out_type=jax.ShapeDtypeStruct((NC, ACC_ROWS, DH), jnp.float32),
        mesh=mesh,
        compiler_params=pltpu.CompilerParams(
            needs_layout_passes=False, use_tc_tiling_on_sc=False),
        scratch_types=[
            pltpu.VMEM((CHUNKS_PER_W, CHUNK), jnp.int32),
            pltpu.VMEM((CHUNKS_PER_W, CHUNK), jnp.int32),
            pltpu.VMEM((CHUNKS_PER_W, CHUNK), jnp.int32),
            pltpu.VMEM((NC * N_PAD,), jnp.float32),
            pltpu.VMEM((64,), jnp.float32),
            pltpu.VMEM((CHUNK, DH), jnp.float32),
            pltpu.VMEM_SHARED((ACC_ROWS, DH), jnp.float32),
            pltpu.SemaphoreType.DMA,
        ],
    )
    partials = sc_call(xt_split, s2, r1, src_off, tgt, typ)

    out = pl.pallas_call(
        _tc_combine,
        grid=(10,),
        in_specs=[pl.BlockSpec((NC, N_NODES // 10, DH), lambda i: (0, i, 0))],
        out_specs=pl.BlockSpec((N_NODES // 10, D), lambda i: (i, 0)),
        out_shape=jax.ShapeDtypeStruct((N_NODES, D), jnp.float32),
    )(partials)
    return out


def kernel(x, edge_index, edge_type, rel_emb, W_lin, W_attn):
    return _run(x, edge_index, edge_type, rel_emb, W_lin, W_attn)


# speedup vs baseline: 11.1236x; 1.2126x over previous
import jax
import jax.numpy as jnp
from jax import lax
from jax.experimental import pallas as pl
from jax.experimental.pallas import tpu as pltpu
from jax.experimental.pallas import tpu_sc as plsc

N_NODES = 10000
N_EDGES = 320000
D = 128
DH = D // 2
N_REL = 50

NC = 2
NS = 16
CHUNK = 128
CHUNKS_PER_W = 159
EPW = CHUNKS_PER_W * CHUNK
E_PAD = NS * EPW
N_PAD = 10016
ACC_ROWS = 10240
ROWS_PER_TILE = ACC_ROWS // NS


def _tc_prep(x_ref, wl_ref, wa_ref, rel_ref, xt_ref, s_ref, r_ref):
    xv = x_ref[...]
    xt = lax.dot_general(
        xv, wl_ref[...], (((1,), (1,)), ((), ())),
        preferred_element_type=jnp.float32)
    xt_ref[0:N_PAD, :] = xt[:, 0:DH]
    xt_ref[N_PAD:2 * N_PAD, :] = xt[:, DH:D]
    wa = wa_ref[...]
    s_ref[...] = lax.dot_general(
        xv, wa, (((1,), (1,)), ((), ())), preferred_element_type=jnp.float32)
    r_ref[...] = lax.dot_general(
        rel_ref[...], wa, (((1,), (1,)), ((), ())),
        preferred_element_type=jnp.float32)


def _tc_combine(p_ref, o_ref):
    o_ref[...] = jnp.maximum(
        jnp.concatenate([p_ref[0], p_ref[1]], axis=1), 0.0)


def _sc_edges(xt_hbm, s_hbm, r_hbm, src_hbm, tgt_hbm, typ_hbm, part_hbm,
              src_v, tgt_v, s_v, r_v, typb,
              rows0, rows1, rows2, acc,
              gsem0, gsem1, gsem2, ssem0, ssem1, ssem2,
              tsem0, tsem1, tsem2):
    c = lax.axis_index("c")
    s = lax.axis_index("s")
    bufs = (rows0, rows1, rows2)
    gsems = (gsem0, gsem1, gsem2)
    ssems = (ssem0, ssem1, ssem2)
    tsems = (tsem0, tsem1, tsem2)

    @pl.loop(0, CHUNK)
    def _zero_rows(i):
        zero16 = jnp.zeros((16,), jnp.float32)
        for h in range(DH // 16):
            rows0[i, pl.ds(h * 16, 16)] = zero16

    for b in range(ROWS_PER_TILE // CHUNK):
        pltpu.sync_copy(rows0, acc.at[pl.ds((s * 5 + b) * CHUNK, CHUNK)])
    plsc.subcore_barrier()

    pltpu.sync_copy(src_hbm.at[c, s], src_v)
    pltpu.sync_copy(tgt_hbm.at[s], tgt_v)
    pltpu.sync_copy(s_hbm, s_v)
    pltpu.sync_copy(r_hbm, r_v)

    coff = c * N_PAD

    def _scale(j, rows_x, X):
        @pl.loop(0, CHUNK // 16)
        def _grp(k):
            sl = pl.ds(k * 16, 16)
            idx16 = src_v[j, sl] - coff
            typ16 = typb[X, sl]
            sv = plsc.load_gather(s_v, [idx16])
            rv = plsc.load_gather(r_v, [typ16])
            a16 = 1.0 / (1.0 + jnp.exp(-(sv + rv)))
            base = k * 16
            for l in range(16):
                a = lax.broadcast_in_dim(a16[l], (16,), ())
                for h in range(DH // 16):
                    fsl = pl.ds(h * 16, 16)
                    rows_x[base + l, fsl] = rows_x[base + l, fsl] * a

    pltpu.async_copy(xt_hbm.at[src_v.at[0]], rows0, gsem0)
    pltpu.async_copy(xt_hbm.at[src_v.at[1]], rows1, gsem1)
    pltpu.async_copy(typ_hbm.at[s, 0], typb.at[0], tsem0)
    pltpu.async_copy(typ_hbm.at[s, 1], typb.at[1], tsem1)

    @pl.loop(0, CHUNKS_PER_W, step=3)
    def _t(t):
        for i in range(3):
            j = t + i
            X = i
            Z = (i + 2) % 3
            pltpu.make_async_copy(
                xt_hbm.at[src_v.at[j]], bufs[X], gsems[X]).wait()
            pltpu.make_async_copy(
                typ_hbm.at[s, j], typb.at[X], tsems[X]).wait()
            _scale(j, bufs[X], X)
            if i == 0:
                @pl.when(t >= 1)
                def _():
                    pltpu.make_async_copy(
                        bufs[Z], acc.at[tgt_v.at[j - 1]], ssems[Z]).wait()
                pltpu.async_copy(
                    xt_hbm.at[src_v.at[j + 2]], bufs[Z], gsems[Z])
                pltpu.async_copy(typ_hbm.at[s, j + 2], typb.at[Z], tsems[Z])
            else:
                pltpu.make_async_copy(
                    bufs[Z], acc.at[tgt_v.at[j - 1]], ssems[Z]).wait()

                @pl.when(j + 2 < CHUNKS_PER_W)
                def _():
                    pltpu.async_copy(
                        xt_hbm.at[src_v.at[j + 2]], bufs[Z], gsems[Z])
                    pltpu.async_copy(
                        typ_hbm.at[s, j + 2], typb.at[Z], tsems[Z])
            pltpu.async_copy(bufs[X], acc.at[tgt_v.at[j]], ssems[X], add=True)

    pltpu.make_async_copy(
        bufs[2], acc.at[tgt_v.at[CHUNKS_PER_W - 1]], ssems[2]).wait()

    plsc.subcore_barrier()
    pltpu.sync_copy(acc.at[pl.ds(s * ROWS_PER_TILE, ROWS_PER_TILE)],
                    part_hbm.at[c, pl.ds(s * ROWS_PER_TILE, ROWS_PER_TILE)])


@jax.jit
def _run(x, edge_index, edge_type, rel_emb, W_lin, W_attn):
    src = edge_index[0].astype(jnp.int32)
    tgt = edge_index[1].astype(jnp.int32)
    typ = edge_type.astype(jnp.int32)

    pad = E_PAD - N_EDGES
    src = jnp.concatenate([src, jnp.full((pad,), N_NODES, jnp.int32)])
    tgt = jnp.concatenate([tgt, jnp.zeros((pad,), jnp.int32)])
    typ = jnp.concatenate([typ, jnp.zeros((pad,), jnp.int32)])
    src = src.reshape(NS, CHUNKS_PER_W, CHUNK)
    tgt = tgt.reshape(NS, CHUNKS_PER_W, CHUNK)
    typ = typ.reshape(NS, CHUNKS_PER_W, CHUNK)
    src_off = src[None] + (jnp.arange(NC, dtype=jnp.int32) * N_PAD)[
        :, None, None, None]

    x_pad = jnp.concatenate(
        [x, jnp.zeros((N_PAD - N_NODES, D), jnp.float32)], axis=0)
    rel_pad = jnp.concatenate(
        [rel_emb, jnp.zeros((64 - N_REL, D), jnp.float32)], axis=0)

    xt_split, s_pad, r_pad = pl.pallas_call(
        _tc_prep,
        out_shape=[
            jax.ShapeDtypeStruct((NC * N_PAD, DH), jnp.float32),
            jax.ShapeDtypeStruct((N_PAD, 1), jnp.float32),
            jax.ShapeDtypeStruct((64, 1), jnp.float32),
        ],
    )(x_pad, W_lin, W_attn, rel_pad)

    s1 = s_pad.reshape(N_PAD)
    r1 = r_pad.reshape(64)

    mesh = plsc.VectorSubcoreMesh(
        core_axis_name="c", subcore_axis_name="s",
        num_cores=NC, num_subcores=NS)
    sc_call = pl.kernel(
        _sc_edges,
        out_type=jax.ShapeDtypeStruct((NC, ACC_ROWS, DH), jnp.float32),
        mesh=mesh,
        compiler_params=pltpu.CompilerParams(
            needs_layout_passes=False, use_tc_tiling_on_sc=False),
        scratch_types=[
            pltpu.VMEM((CHUNKS_PER_W, CHUNK), jnp.int32),
            pltpu.VMEM((CHUNKS_PER_W, CHUNK), jnp.int32),
            pltpu.VMEM((N_PAD,), jnp.float32),
            pltpu.VMEM((64,), jnp.float32),
            pltpu.VMEM((3, CHUNK), jnp.int32),
            pltpu.VMEM((CHUNK, DH), jnp.float32),
            pltpu.VMEM((CHUNK, DH), jnp.float32),
            pltpu.VMEM((CHUNK, DH), jnp.float32),
            pltpu.VMEM_SHARED((ACC_ROWS, DH), jnp.float32),
            pltpu.SemaphoreType.DMA,
            pltpu.SemaphoreType.DMA,
            pltpu.SemaphoreType.DMA,
            pltpu.SemaphoreType.DMA,
            pltpu.SemaphoreType.DMA,
            pltpu.SemaphoreType.DMA,
            pltpu.SemaphoreType.DMA,
            pltpu.SemaphoreType.DMA,
            pltpu.SemaphoreType.DMA,
        ],
    )
    partials = sc_call(xt_split, s1, r1, src_off, tgt, typ)

    out = pl.pallas_call(
        _tc_combine,
        grid=(10,),
        in_specs=[pl.BlockSpec((NC, N_NODES // 10, DH), lambda i: (0, i, 0))],
        out_specs=pl.BlockSpec((N_NODES // 10, D), lambda i: (i, 0)),
        out_shape=jax.ShapeDtypeStruct((N_NODES, D), jnp.float32),
    )(partials)
    return out


def kernel(x, edge_index, edge_type, rel_emb, W_lin, W_attn):
    return _run(x, edge_index, edge_type, rel_emb, W_lin, W_attn)
